# baseline (speedup 1.0000x reference)
import functools

import jax
import jax.numpy as jnp
from jax import lax
from jax.experimental import pallas as pl
from jax.experimental.pallas import tpu as pltpu
from jax.experimental.pallas import tpu_sc as plsc

B = 16384
D = 64
LAM = 1e-05
NC = 2
NS = 16
NW = NC * NS
BPW = B // NW
CHUNK = 128
NCHUNK = BPW // CHUNK

_mesh = plsc.VectorSubcoreMesh(core_axis_name="c", subcore_axis_name="s")


@functools.partial(
    pl.kernel,
    mesh=_mesh,
    compiler_params=pltpu.CompilerParams(
        needs_layout_passes=False, use_tc_tiling_on_sc=False),
    out_type=[
        jax.ShapeDtypeStruct((B,), jnp.float32),
        jax.ShapeDtypeStruct((NW, 16), jnp.float32),
    ],
    scratch_types=[
        pltpu.VMEM((BPW,), jnp.int32),
        pltpu.VMEM((BPW,), jnp.int32),
        pltpu.VMEM((BPW,), jnp.int32),
        pltpu.VMEM((BPW,), jnp.int32),
        pltpu.VMEM((CHUNK, D), jnp.float32),
        pltpu.VMEM((CHUNK, D), jnp.float32),
        pltpu.VMEM((CHUNK, D), jnp.float32),
        pltpu.VMEM((CHUNK, D), jnp.float32),
        pltpu.VMEM((CHUNK, D), jnp.float32),
        pltpu.VMEM((BPW,), jnp.float32),
        pltpu.VMEM((16,), jnp.float32),
        pltpu.SemaphoreType.DMA,
    ],
)
def _transh_sc(h_hbm, r_hbm, pt_hbm, nt_hbm, ent_hbm, rel_hbm, nrm_hbm,
               d_hbm, l2_hbm,
               hi_v, ri_v, pi_v, ti_v,
               hrow, rrow, nrow, prow, trow, dbuf, l2buf, sem):
    wid = lax.axis_index("s") * NC + lax.axis_index("c")
    base = wid * BPW

    pltpu.sync_copy(h_hbm.at[pl.ds(base, BPW)], hi_v)
    pltpu.sync_copy(r_hbm.at[pl.ds(base, BPW)], ri_v)
    pltpu.sync_copy(pt_hbm.at[pl.ds(base, BPW)], pi_v)
    pltpu.sync_copy(nt_hbm.at[pl.ds(base, BPW)], ti_v)

    lanes = jnp.arange(16, dtype=jnp.int32)
    l2acc = jnp.zeros((16,), jnp.float32)
    dvec0 = jnp.zeros((16,), jnp.float32)
    for c in range(NCHUNK):
        sl = pl.ds(c * CHUNK, CHUNK)
        cps = [
            pltpu.async_copy(ent_hbm.at[hi_v.at[sl]], hrow, sem),
            pltpu.async_copy(rel_hbm.at[ri_v.at[sl]], rrow, sem),
            pltpu.async_copy(nrm_hbm.at[ri_v.at[sl]], nrow, sem),
            pltpu.async_copy(ent_hbm.at[pi_v.at[sl]], prow, sem),
            pltpu.async_copy(ent_hbm.at[ti_v.at[sl]], trow, sem),
        ]
        for cp in cps:
            cp.wait()

        def row_body(i, carry, c=c):
            l2a, dvec = carry
            n0 = nrow[i, pl.ds(0, 16)]
            n1 = nrow[i, pl.ds(16, 16)]
            n2 = nrow[i, pl.ds(32, 16)]
            n3 = nrow[i, pl.ds(48, 16)]
            h0 = hrow[i, pl.ds(0, 16)]
            h1 = hrow[i, pl.ds(16, 16)]
            h2 = hrow[i, pl.ds(32, 16)]
            h3 = hrow[i, pl.ds(48, 16)]
            p0 = prow[i, pl.ds(0, 16)]
            p1 = prow[i, pl.ds(16, 16)]
            p2 = prow[i, pl.ds(32, 16)]
            p3 = prow[i, pl.ds(48, 16)]
            t0 = trow[i, pl.ds(0, 16)]
            t1 = trow[i, pl.ds(16, 16)]
            t2 = trow[i, pl.ds(32, 16)]
            t3 = trow[i, pl.ds(48, 16)]
            r0 = rrow[i, pl.ds(0, 16)]
            r1 = rrow[i, pl.ds(16, 16)]
            r2 = rrow[i, pl.ds(32, 16)]
            r3 = rrow[i, pl.ds(48, 16)]

            nsq = jnp.broadcast_to(
                jnp.sum(n0 * n0 + n1 * n1 + n2 * n2 + n3 * n3), (16,))
            inv = 1.0 / jnp.maximum(nsq, 1e-24)
            hn = jnp.broadcast_to(
                jnp.sum(h0 * n0 + h1 * n1 + h2 * n2 + h3 * n3), (16,))
            pn = jnp.broadcast_to(
                jnp.sum(p0 * n0 + p1 * n1 + p2 * n2 + p3 * n3), (16,))
            tn = jnp.broadcast_to(
                jnp.sum(t0 * n0 + t1 * n1 + t2 * n2 + t3 * n3), (16,))
            ah = hn * inv
            ap = pn * inv
            at = tn * inv

            ht0 = h0 - ah * n0
            ht1 = h1 - ah * n1
            ht2 = h2 - ah * n2
            ht3 = h3 - ah * n3
            pt0 = p0 - ap * n0
            pt1 = p1 - ap * n1
            pt2 = p2 - ap * n2
            pt3 = p3 - ap * n3
            tt0 = t0 - at * n0
            tt1 = t1 - at * n1
            tt2 = t2 - at * n2
            tt3 = t3 - at * n3

            b0 = ht0 + r0
            b1 = ht1 + r1
            b2 = ht2 + r2
            b3 = ht3 + r3
            pd0 = b0 - pt0
            pd1 = b1 - pt1
            pd2 = b2 - pt2
            pd3 = b3 - pt3
            nd0 = b0 - tt0
            nd1 = b1 - tt1
            nd2 = b2 - tt2
            nd3 = b3 - tt3
            pos = jnp.sum(pd0 * pd0 + pd1 * pd1 + pd2 * pd2 + pd3 * pd3)
            neg = jnp.sum(nd0 * nd0 + nd1 * nd1 + nd2 * nd2 + nd3 * nd3)
            dvec = jnp.where(lanes == (i & 15), neg - pos, dvec)

            @pl.when((i & 15) == 15)
            def _():
                dbuf[pl.ds(c * CHUNK + i - 15, 16)] = dvec

            l2a = l2a + (ht0 * ht0 + ht1 * ht1 + ht2 * ht2 + ht3 * ht3)
            l2a = l2a + (r0 * r0 + r1 * r1 + r2 * r2 + r3 * r3)
            l2a = l2a + (pt0 * pt0 + pt1 * pt1 + pt2 * pt2 + pt3 * pt3)
            l2a = l2a + (tt0 * tt0 + tt1 * tt1 + tt2 * tt2 + tt3 * tt3)
            return l2a, dvec

        l2acc, dvec0 = lax.fori_loop(0, CHUNK, row_body, (l2acc, dvec0))

    l2buf[...] = l2acc
    pltpu.sync_copy(dbuf, d_hbm.at[pl.ds(base, BPW)])
    pltpu.sync_copy(l2buf, l2_hbm.at[wid])


def _finish_body(d_ref, l2_ref, o_ref):
    d = d_ref[...]
    sp = jnp.maximum(-d, 0.0) + jnp.log(1.0 + jnp.exp(-jnp.abs(d)))
    triplet = jnp.sum(sp) / B
    l2 = jnp.sum(l2_ref[...]) / (2.0 * B)
    o_ref[...] = jnp.reshape(triplet + LAM * l2, (1, 1))


def kernel(h, r, pos_t, neg_t, entity_w, relation_w, norm_w):
    h = h.astype(jnp.int32)
    r = r.astype(jnp.int32)
    pos_t = pos_t.astype(jnp.int32)
    neg_t = neg_t.astype(jnp.int32)
    d, l2 = _transh_sc(h, r, pos_t, neg_t, entity_w, relation_w, norm_w)
    out = pl.pallas_call(
        _finish_body,
        out_shape=jax.ShapeDtypeStruct((1, 1), jnp.float32),
    )(d.reshape(128, 128), l2)
    return out[0, 0]


# speedup vs baseline: 1.0121x; 1.0121x over previous
import functools

import jax
import jax.numpy as jnp
from jax import lax
from jax.experimental import pallas as pl
from jax.experimental.pallas import tpu as pltpu
from jax.experimental.pallas import tpu_sc as plsc

B = 16384
D = 64
LAM = 1e-05
NC = 2
NS = 16
NW = NC * NS
BPW = B // NW
CHUNK = 128
NCHUNK = BPW // CHUNK

_mesh = plsc.VectorSubcoreMesh(core_axis_name="c", subcore_axis_name="s")


@functools.partial(
    pl.kernel,
    mesh=_mesh,
    compiler_params=pltpu.CompilerParams(
        needs_layout_passes=False, use_tc_tiling_on_sc=False),
    out_type=[
        jax.ShapeDtypeStruct((B,), jnp.float32),
        jax.ShapeDtypeStruct((NW, 16), jnp.float32),
    ],
    scratch_types=[
        pltpu.VMEM((BPW,), jnp.int32),
        pltpu.VMEM((BPW,), jnp.int32),
        pltpu.VMEM((BPW,), jnp.int32),
        pltpu.VMEM((BPW,), jnp.int32),
        pltpu.VMEM((CHUNK, D), jnp.float32),
        pltpu.VMEM((CHUNK, D), jnp.float32),
        pltpu.VMEM((CHUNK, D), jnp.float32),
        pltpu.VMEM((CHUNK, D), jnp.float32),
        pltpu.VMEM((CHUNK, D), jnp.float32),
        pltpu.VMEM((CHUNK, D), jnp.float32),
        pltpu.VMEM((CHUNK, D), jnp.float32),
        pltpu.VMEM((CHUNK, D), jnp.float32),
        pltpu.VMEM((CHUNK, D), jnp.float32),
        pltpu.VMEM((CHUNK, D), jnp.float32),
        pltpu.VMEM((BPW,), jnp.float32),
        pltpu.VMEM((16,), jnp.float32),
        pltpu.SemaphoreType.DMA,
        pltpu.SemaphoreType.DMA,
        pltpu.SemaphoreType.DMA,
    ],
)
def _transh_sc(h_hbm, r_hbm, pt_hbm, nt_hbm, ent_hbm, rel_hbm, nrm_hbm,
               d_hbm, l2_hbm,
               hi_v, ri_v, pi_v, ti_v,
               hrow0, rrow0, nrow0, prow0, trow0,
               hrow1, rrow1, nrow1, prow1, trow1,
               dbuf, l2buf, isem, sem0, sem1):
    wid = lax.axis_index("s") * NC + lax.axis_index("c")
    base = wid * BPW

    icps = [
        pltpu.async_copy(h_hbm.at[pl.ds(base, BPW)], hi_v, isem),
        pltpu.async_copy(r_hbm.at[pl.ds(base, BPW)], ri_v, isem),
        pltpu.async_copy(pt_hbm.at[pl.ds(base, BPW)], pi_v, isem),
        pltpu.async_copy(nt_hbm.at[pl.ds(base, BPW)], ti_v, isem),
    ]
    for cp in icps:
        cp.wait()

    bufs = [(hrow0, rrow0, nrow0, prow0, trow0),
            (hrow1, rrow1, nrow1, prow1, trow1)]
    sems = [sem0, sem1]

    def fire(c):
        sl = pl.ds(c * CHUNK, CHUNK)
        hrow, rrow, nrow, prow, trow = bufs[c % 2]
        sem = sems[c % 2]
        return [
            pltpu.async_copy(ent_hbm.at[hi_v.at[sl]], hrow, sem),
            pltpu.async_copy(rel_hbm.at[ri_v.at[sl]], rrow, sem),
            pltpu.async_copy(nrm_hbm.at[ri_v.at[sl]], nrow, sem),
            pltpu.async_copy(ent_hbm.at[pi_v.at[sl]], prow, sem),
            pltpu.async_copy(ent_hbm.at[ti_v.at[sl]], trow, sem),
        ]

    lanes = jnp.arange(16, dtype=jnp.int32)
    l2acc = jnp.zeros((16,), jnp.float32)
    dvec0 = jnp.zeros((16,), jnp.float32)
    pending = {0: fire(0)}
    for c in range(NCHUNK):
        if c + 1 < NCHUNK:
            pending[c + 1] = fire(c + 1)
        for cp in pending.pop(c):
            cp.wait()
        hrow, rrow, nrow, prow, trow = bufs[c % 2]

        def row_body(i, carry, c=c, hrow=hrow, rrow=rrow, nrow=nrow,
                     prow=prow, trow=trow):
            l2a, dvec = carry
            n0 = nrow[i, pl.ds(0, 16)]
            n1 = nrow[i, pl.ds(16, 16)]
            n2 = nrow[i, pl.ds(32, 16)]
            n3 = nrow[i, pl.ds(48, 16)]
            h0 = hrow[i, pl.ds(0, 16)]
            h1 = hrow[i, pl.ds(16, 16)]
            h2 = hrow[i, pl.ds(32, 16)]
            h3 = hrow[i, pl.ds(48, 16)]
            p0 = prow[i, pl.ds(0, 16)]
            p1 = prow[i, pl.ds(16, 16)]
            p2 = prow[i, pl.ds(32, 16)]
            p3 = prow[i, pl.ds(48, 16)]
            t0 = trow[i, pl.ds(0, 16)]
            t1 = trow[i, pl.ds(16, 16)]
            t2 = trow[i, pl.ds(32, 16)]
            t3 = trow[i, pl.ds(48, 16)]
            r0 = rrow[i, pl.ds(0, 16)]
            r1 = rrow[i, pl.ds(16, 16)]
            r2 = rrow[i, pl.ds(32, 16)]
            r3 = rrow[i, pl.ds(48, 16)]

            nsq = jnp.broadcast_to(
                jnp.sum(n0 * n0 + n1 * n1 + n2 * n2 + n3 * n3), (16,))
            inv = 1.0 / jnp.maximum(nsq, 1e-24)
            hn = jnp.broadcast_to(
                jnp.sum(h0 * n0 + h1 * n1 + h2 * n2 + h3 * n3), (16,))
            pn = jnp.broadcast_to(
                jnp.sum(p0 * n0 + p1 * n1 + p2 * n2 + p3 * n3), (16,))
            tn = jnp.broadcast_to(
                jnp.sum(t0 * n0 + t1 * n1 + t2 * n2 + t3 * n3), (16,))
            ah = hn * inv
            ap = pn * inv
            at = tn * inv

            ht0 = h0 - ah * n0
            ht1 = h1 - ah * n1
            ht2 = h2 - ah * n2
            ht3 = h3 - ah * n3
            pt0 = p0 - ap * n0
            pt1 = p1 - ap * n1
            pt2 = p2 - ap * n2
            pt3 = p3 - ap * n3
            tt0 = t0 - at * n0
            tt1 = t1 - at * n1
            tt2 = t2 - at * n2
            tt3 = t3 - at * n3

            b0 = ht0 + r0
            b1 = ht1 + r1
            b2 = ht2 + r2
            b3 = ht3 + r3
            pd0 = b0 - pt0
            pd1 = b1 - pt1
            pd2 = b2 - pt2
            pd3 = b3 - pt3
            nd0 = b0 - tt0
            nd1 = b1 - tt1
            nd2 = b2 - tt2
            nd3 = b3 - tt3
            pos = jnp.sum(pd0 * pd0 + pd1 * pd1 + pd2 * pd2 + pd3 * pd3)
            neg = jnp.sum(nd0 * nd0 + nd1 * nd1 + nd2 * nd2 + nd3 * nd3)
            dvec = jnp.where(lanes == (i & 15), neg - pos, dvec)

            @pl.when((i & 15) == 15)
            def _():
                dbuf[pl.ds(c * CHUNK + i - 15, 16)] = dvec

            l2a = l2a + (ht0 * ht0 + ht1 * ht1 + ht2 * ht2 + ht3 * ht3)
            l2a = l2a + (r0 * r0 + r1 * r1 + r2 * r2 + r3 * r3)
            l2a = l2a + (pt0 * pt0 + pt1 * pt1 + pt2 * pt2 + pt3 * pt3)
            l2a = l2a + (tt0 * tt0 + tt1 * tt1 + tt2 * tt2 + tt3 * tt3)
            return l2a, dvec

        l2acc, dvec0 = lax.fori_loop(0, CHUNK, row_body, (l2acc, dvec0),
                                     unroll=4)

    l2buf[...] = l2acc
    pltpu.sync_copy(dbuf, d_hbm.at[pl.ds(base, BPW)])
    pltpu.sync_copy(l2buf, l2_hbm.at[wid])


def _finish_body(d_ref, l2_ref, o_ref):
    d = d_ref[...]
    sp = jnp.maximum(-d, 0.0) + jnp.log(1.0 + jnp.exp(-jnp.abs(d)))
    triplet = jnp.sum(sp) / B
    l2 = jnp.sum(l2_ref[...]) / (2.0 * B)
    o_ref[...] = jnp.reshape(triplet + LAM * l2, (1, 1))


def kernel(h, r, pos_t, neg_t, entity_w, relation_w, norm_w):
    h = h.astype(jnp.int32)
    r = r.astype(jnp.int32)
    pos_t = pos_t.astype(jnp.int32)
    neg_t = neg_t.astype(jnp.int32)
    d, l2 = _transh_sc(h, r, pos_t, neg_t, entity_w, relation_w, norm_w)
    out = pl.pallas_call(
        _finish_body,
        out_shape=jax.ShapeDtypeStruct((1, 1), jnp.float32),
    )(d.reshape(128, 128), l2)
    return out[0, 0]
